# Initial kernel scaffold; baseline (speedup 1.0000x reference)
import jax
import jax.numpy as jnp
from jax.experimental import pallas as pl


def kernel(y_pred, y_true, cluster_ids):
    raise NotImplementedError("write your pallas kernel here")


# speedup vs baseline: 3.3737x; 3.3737x over previous
import functools

import jax
import jax.numpy as jnp
from jax import lax
from jax.experimental import pallas as pl
from jax.experimental.pallas import tpu as pltpu
from jax.experimental.pallas import tpu_sc as plsc

N = 3200000
NUM_CLASSES = 5
S = 10000

NC = 2
NS = 16
NW = NC * NS
L = 16

ROWS_PER_TILE = N // NW
R = 4000
NBLK = ROWS_PER_TILE // R
VEC_PER_BLK = R // L

_LN2 = 0.6931471805599453
_SQRT2 = 1.4142135381698608


def _log_small(s):
    bits = plsc.bitcast(s, jnp.int32)
    e = (bits >> 23) - 127
    m = plsc.bitcast((bits & 0x007FFFFF) | 0x3F800000, jnp.float32)
    adj = m > _SQRT2
    m = jnp.where(adj, m * 0.5, m)
    ef = e.astype(jnp.float32) + jnp.where(adj, 1.0, 0.0)
    t = (m - 1.0) / (m + 1.0)
    t2 = t * t
    p = t * (2.0 + t2 * (2.0 / 3.0 + t2 * (2.0 / 5.0 + t2 * (2.0 / 7.0 + t2 * (2.0 / 9.0)))))
    return ef * _LN2 + p


_mesh = plsc.VectorSubcoreMesh(
    core_axis_name="c", subcore_axis_name="s", num_cores=NC, num_subcores=NS
)


@functools.partial(
    pl.kernel,
    out_type=(
        jax.ShapeDtypeStruct((NW, S), jnp.float32),
        jax.ShapeDtypeStruct((NW, S), jnp.float32),
    ),
    mesh=_mesh,
    compiler_params=pltpu.CompilerParams(needs_layout_passes=False),
    scratch_types=(
        pltpu.VMEM((R * NUM_CLASSES,), jnp.float32),
        pltpu.VMEM((R,), jnp.int32),
        pltpu.VMEM((R,), jnp.int32),
        pltpu.VMEM((S,), jnp.float32),
        pltpu.VMEM((S,), jnp.float32),
    ),
)
def _sc_partials(yp_hbm, yt_hbm, ids_hbm, psum_hbm, pcnt_hbm,
                 yp_v, yt_v, ids_v, sum_v, cnt_v):
    wid = lax.axis_index("s") * NC + lax.axis_index("c")
    row0 = wid * ROWS_PER_TILE

    zeros = jnp.zeros((L,), jnp.float32)

    @pl.loop(0, S // L)
    def _zero(i):
        sum_v[pl.ds(i * L, L)] = zeros
        cnt_v[pl.ds(i * L, L)] = zeros

    iota = lax.iota(jnp.int32, L)
    iota5 = iota * NUM_CLASSES
    ones = jnp.ones((L,), jnp.float32)

    @pl.loop(0, NBLK)
    def _blk(b):
        r0 = row0 + b * R
        pltpu.sync_copy(yp_hbm.at[pl.ds(r0 * NUM_CLASSES, R * NUM_CLASSES)], yp_v)
        pltpu.sync_copy(yt_hbm.at[pl.ds(r0, R)], yt_v)
        pltpu.sync_copy(ids_hbm.at[pl.ds(r0, R)], ids_v)

        @pl.loop(0, VEC_PER_BLK)
        def _vec(j):
            base = j * (L * NUM_CLASSES)
            idx0 = base + iota5
            c0 = plsc.load_gather(yp_v, [idx0])
            c1 = plsc.load_gather(yp_v, [idx0 + 1])
            c2 = plsc.load_gather(yp_v, [idx0 + 2])
            c3 = plsc.load_gather(yp_v, [idx0 + 3])
            c4 = plsc.load_gather(yp_v, [idx0 + 4])
            m = jnp.maximum(jnp.maximum(jnp.maximum(c0, c1), jnp.maximum(c2, c3)), c4)
            ssum = (jnp.exp(c0 - m) + jnp.exp(c1 - m)) + (
                jnp.exp(c2 - m) + jnp.exp(c3 - m)) + jnp.exp(c4 - m)
            yt = yt_v[pl.ds(j * L, L)]
            picked = plsc.load_gather(yp_v, [idx0 + yt])
            loss = m + _log_small(ssum) - picked
            seg = ids_v[pl.ds(j * L, L)]
            plsc.addupdate_scatter(sum_v, [seg], loss)
            plsc.addupdate_scatter(cnt_v, [seg], ones)

    pltpu.sync_copy(sum_v, psum_hbm.at[wid])
    pltpu.sync_copy(cnt_v, pcnt_hbm.at[wid])


def _tc_combine_body(ps_ref, pc_ref, out_ref):
    out_ref[...] = jnp.sum(ps_ref[...], axis=0) / jnp.sum(pc_ref[...], axis=0)


def _tc_combine(psum, pcnt):
    return pl.pallas_call(
        _tc_combine_body,
        out_shape=jax.ShapeDtypeStruct((S,), jnp.float32),
    )(psum, pcnt)


def kernel(y_pred, y_true, cluster_ids):
    yp_flat = y_pred.reshape(-1)
    ids = cluster_ids.reshape(-1)
    psum, pcnt = _sc_partials(yp_flat, y_true, ids)
    return _tc_combine(psum, pcnt)
